# Initial kernel scaffold; baseline (speedup 1.0000x reference)
import jax
import jax.numpy as jnp
from jax.experimental import pallas as pl


def kernel(d, W1, W2a, g2a, b2a, W2b, W3a, g3a, b3a, W3b, g3b, b3b, W3c, gf, bf):
    raise NotImplementedError("write your pallas kernel here")


# speedup vs baseline: 3.0202x; 3.0202x over previous
import jax
import jax.numpy as jnp
from jax.experimental import pallas as pl
from jax.experimental.pallas import tpu as pltpu

_B = 4
_L = 67600
_C0 = 522

_CP = pltpu.CompilerParams(
    dimension_semantics=("arbitrary",),
    vmem_limit_bytes=64 * 1024 * 1024,
)
_CP2 = pltpu.CompilerParams(
    dimension_semantics=("arbitrary", "arbitrary"),
    vmem_limit_bytes=64 * 1024 * 1024,
)


def _kpool(d_ref, f_ref):
    d = d_ref[0, 0]
    n = jnp.maximum(d, 0.0)
    npad = jnp.pad(n, ((1, 1), (1, 1)))
    v = jnp.maximum(jnp.maximum(npad[0:512], npad[1:513]), npad[2:514])
    v2 = v.reshape(256, 2, 514)[:, 0, :]
    h = jnp.maximum(jnp.maximum(v2[:, 0:512], v2[:, 1:513]), v2[:, 2:514])
    r = jax.lax.broadcasted_iota(jnp.int32, (512, 256), 0)
    c = jax.lax.broadcasted_iota(jnp.int32, (512, 256), 1)
    sel = (r == 2 * c).astype(jnp.float32)
    f_ref[0] = jax.lax.dot(h, sel, preferred_element_type=jnp.float32)


def _k1b(f_ref, w_ref, y_ref, mm_ref, st_ref):
    fv = f_ref[0]
    mask = (fv > 0.0).astype(jnp.float32)

    @pl.when(pl.program_id(1) == 0)
    def _():
        mm_ref[0] = mask

    rolls = [pltpu.roll(fv, (_C0 - (dy * 260 + dx)) % _L, axis=1)
             for dy in range(5) for dx in range(5)]
    X = jnp.concatenate(rolls, axis=0)
    Y = jax.lax.dot(w_ref[0], X, preferred_element_type=jnp.float32)
    Y = Y * mask
    y_ref[0] = Y
    s1 = jnp.sum(Y, axis=1, keepdims=True)
    s2 = jnp.sum(Y * Y, axis=1, keepdims=True)
    cnt = jnp.full((40, 1), jnp.sum(mask), jnp.float32)
    z = jnp.zeros((40, 5), jnp.float32)
    st_ref[0, 0] = jnp.concatenate([s1, s2, cnt, z], axis=1)


def _conv_body(y_ref, mm_ref, ac_ref, w_ref):
    y = y_ref[0]
    mask = mm_ref[0]
    a = ac_ref[:, 0:1]
    c = ac_ref[:, 1:2]
    h = jnp.maximum(a * y + c, 0.0) * mask
    w = w_ref[0]
    cout = w.shape[1]
    acc = jnp.zeros((cout, _L), jnp.float32)
    for t in range(25):
        sh = (_C0 - ((t // 5) * 260 + t % 5)) % _L
        r = pltpu.roll(h, sh, axis=1)
        acc = acc + jax.lax.dot(w[t], r, preferred_element_type=jnp.float32)
    return acc, mask


def _kconv(y_ref, mm_ref, ac_ref, w_ref, out_ref):
    acc, _ = _conv_body(y_ref, mm_ref, ac_ref, w_ref)
    out_ref[0] = acc


def _kconv_st(y_ref, mm_ref, ac_ref, w_ref, out_ref, st_ref):
    acc, mask = _conv_body(y_ref, mm_ref, ac_ref, w_ref)
    out = acc * mask
    out_ref[0] = out
    s1 = jnp.sum(out, axis=1, keepdims=True)
    s2 = jnp.sum(out * out, axis=1, keepdims=True)
    cnt = jnp.full((out.shape[0], 1), jnp.sum(mask), jnp.float32)
    z = jnp.zeros((out.shape[0], 5), jnp.float32)
    st_ref[0, 0] = jnp.concatenate([s1, s2, cnt, z], axis=1)


def _k3b(x1_ref, x2_ref, x3_ref, mm_ref, s_ref, st_ref):
    s = x1_ref[0] + x2_ref[0] + x3_ref[0]
    mask = mm_ref[0]
    s_ref[0] = s
    sm = s * mask
    s1 = jnp.sum(sm, axis=1, keepdims=True)
    s2 = jnp.sum(sm * sm, axis=1, keepdims=True)
    cnt = jnp.full((16, 1), jnp.sum(mask), jnp.float32)
    z = jnp.zeros((16, 5), jnp.float32)
    st_ref[0] = jnp.concatenate([s1, s2, cnt, z], axis=1)


def _k4(s_ref, mm_ref, ac_ref, o_ref):
    s = s_ref[0]
    o_ref[0] = (jnp.maximum(ac_ref[:, 0:1] * s + ac_ref[:, 1:2], 0.0)
                * mm_ref[0])


def _bn_params(s, ss, cnt, g, b, eps=1e-5):
    cnt = jnp.maximum(cnt, 1.0)
    mean = s / cnt
    var = ss / cnt - mean * mean
    a = g * jax.lax.rsqrt(var + eps)
    c = b - mean * a
    return jnp.stack([a, c], axis=1)


def _plane_spec(ch, blk=None, ch_index=None):
    blk = ch if blk is None else blk

    def imap(b, *rest):
        if ch_index is not None:
            return (b, ch_index, 0)
        if rest:
            return (b, rest[0], 0)
        return (b, 0, 0)

    return pl.BlockSpec((1, blk, _L), imap)


def kernel(d, W1, W2a, g2a, b2a, W2b, W3a, g3a, b3a, W3b, g3b, b3b, W3c,
           gf, bf):
    f = pl.pallas_call(
        _kpool,
        grid=(_B,),
        in_specs=[pl.BlockSpec((1, 1, 512, 512), lambda b: (b, 0, 0, 0))],
        out_specs=pl.BlockSpec((1, 256, 256), lambda b: (b, 0, 0)),
        out_shape=jax.ShapeDtypeStruct((_B, 256, 256), jnp.float32),
        compiler_params=_CP,
    )(d)

    fflat = jnp.pad(f, ((0, 0), (2, 2), (2, 2))).reshape(_B, 1, _L)

    w180 = jnp.concatenate([W2a.reshape(25, 32), W3a.reshape(25, 32),
                            W1.reshape(25, 16)], axis=-1)
    w180t = w180.T.reshape(2, 40, 25)

    Y, mm, st1 = pl.pallas_call(
        _k1b,
        grid=(_B, 2),
        in_specs=[
            pl.BlockSpec((1, 1, _L), lambda b, co: (b, 0, 0)),
            pl.BlockSpec((1, 40, 25), lambda b, co: (co, 0, 0)),
        ],
        out_specs=[
            pl.BlockSpec((1, 40, _L), lambda b, co: (b, co, 0)),
            pl.BlockSpec((1, 1, _L), lambda b, co: (b, 0, 0)),
            pl.BlockSpec((1, 1, 40, 8), lambda b, co: (b, co, 0, 0)),
        ],
        out_shape=[
            jax.ShapeDtypeStruct((_B, 80, _L), jnp.float32),
            jax.ShapeDtypeStruct((_B, 1, _L), jnp.float32),
            jax.ShapeDtypeStruct((_B, 2, 40, 8), jnp.float32),
        ],
        compiler_params=_CP2,
    )(fflat, w180t)

    stt = jnp.sum(st1, axis=0)
    cnt = stt[0, 0, 2]
    ac2 = _bn_params(stt[0, 0:32, 0], stt[0, 0:32, 1], cnt, g2a, b2a)
    s_y3 = jnp.concatenate([stt[0, 32:40, 0], stt[1, 0:24, 0]])
    ss_y3 = jnp.concatenate([stt[0, 32:40, 1], stt[1, 0:24, 1]])
    ac3 = _bn_params(s_y3, ss_y3, cnt, g3a, b3a)

    x2 = pl.pallas_call(
        _kconv,
        grid=(_B,),
        in_specs=[
            pl.BlockSpec((1, 32, _L), lambda b: (b, 0, 0)),
            pl.BlockSpec((1, 1, _L), lambda b: (b, 0, 0)),
            pl.BlockSpec((32, 2), lambda b: (0, 0)),
            pl.BlockSpec((1, 25, 16, 32), lambda b: (0, 0, 0, 0)),
        ],
        out_specs=pl.BlockSpec((1, 16, _L), lambda b: (b, 0, 0)),
        out_shape=jax.ShapeDtypeStruct((_B, 16, _L), jnp.float32),
        compiler_params=_CP,
    )(Y, mm, ac2, W2b.reshape(25, 32, 16).transpose(0, 2, 1)[None])

    w3b = W3b.reshape(25, 32, 32).transpose(0, 2, 1)
    w3b = w3b.reshape(25, 2, 16, 32).transpose(1, 0, 2, 3)

    y3b, st3b = pl.pallas_call(
        _kconv_st,
        grid=(_B, 2),
        in_specs=[
            pl.BlockSpec((1, 32, _L), lambda b, co: (b, 1, 0)),
            pl.BlockSpec((1, 1, _L), lambda b, co: (b, 0, 0)),
            pl.BlockSpec((32, 2), lambda b, co: (0, 0)),
            pl.BlockSpec((1, 25, 16, 32), lambda b, co: (co, 0, 0, 0)),
        ],
        out_specs=[
            pl.BlockSpec((1, 16, _L), lambda b, co: (b, co, 0)),
            pl.BlockSpec((1, 1, 16, 8), lambda b, co: (b, co, 0, 0)),
        ],
        out_shape=[
            jax.ShapeDtypeStruct((_B, 32, _L), jnp.float32),
            jax.ShapeDtypeStruct((_B, 2, 16, 8), jnp.float32),
        ],
        compiler_params=_CP2,
    )(Y, mm, ac3, w3b)

    st3 = jnp.sum(st3b, axis=0)
    ac3b = _bn_params(jnp.concatenate([st3[0, :, 0], st3[1, :, 0]]),
                      jnp.concatenate([st3[0, :, 1], st3[1, :, 1]]),
                      cnt, g3b, b3b)

    x3 = pl.pallas_call(
        _kconv,
        grid=(_B,),
        in_specs=[
            pl.BlockSpec((1, 32, _L), lambda b: (b, 0, 0)),
            pl.BlockSpec((1, 1, _L), lambda b: (b, 0, 0)),
            pl.BlockSpec((32, 2), lambda b: (0, 0)),
            pl.BlockSpec((1, 25, 16, 32), lambda b: (0, 0, 0, 0)),
        ],
        out_specs=pl.BlockSpec((1, 16, _L), lambda b: (b, 0, 0)),
        out_shape=jax.ShapeDtypeStruct((_B, 16, _L), jnp.float32),
        compiler_params=_CP,
    )(y3b, mm, ac3b, W3c.reshape(25, 32, 16).transpose(0, 2, 1)[None])

    s, stf = pl.pallas_call(
        _k3b,
        grid=(_B,),
        in_specs=[
            pl.BlockSpec((1, 16, _L), lambda b: (b, 4, 0)),
            pl.BlockSpec((1, 16, _L), lambda b: (b, 0, 0)),
            pl.BlockSpec((1, 16, _L), lambda b: (b, 0, 0)),
            pl.BlockSpec((1, 1, _L), lambda b: (b, 0, 0)),
        ],
        out_specs=[
            pl.BlockSpec((1, 16, _L), lambda b: (b, 0, 0)),
            pl.BlockSpec((1, 16, 8), lambda b: (b, 0, 0)),
        ],
        out_shape=[
            jax.ShapeDtypeStruct((_B, 16, _L), jnp.float32),
            jax.ShapeDtypeStruct((_B, 16, 8), jnp.float32),
        ],
        compiler_params=_CP,
    )(Y, x2, x3, mm)

    stfs = jnp.sum(stf, axis=0)
    acf = _bn_params(stfs[:, 0], stfs[:, 1], cnt, gf, bf)

    o = pl.pallas_call(
        _k4,
        grid=(_B,),
        in_specs=[
            pl.BlockSpec((1, 16, _L), lambda b: (b, 0, 0)),
            pl.BlockSpec((1, 1, _L), lambda b: (b, 0, 0)),
            pl.BlockSpec((16, 2), lambda b: (0, 0)),
        ],
        out_specs=pl.BlockSpec((1, 16, _L), lambda b: (b, 0, 0)),
        out_shape=jax.ShapeDtypeStruct((_B, 16, _L), jnp.float32),
        compiler_params=_CP,
    )(s, mm, acf)

    return o.reshape(_B, 16, 260, 260)[:, :, 2:258, 2:258]


# speedup vs baseline: 3.0202x; 1.0000x over previous
import jax
import jax.numpy as jnp
from jax.experimental import pallas as pl
from jax.experimental.pallas import tpu as pltpu

_B = 4
_L = 67600
_C0 = 522

_CP = pltpu.CompilerParams(
    dimension_semantics=("parallel",),
    vmem_limit_bytes=64 * 1024 * 1024,
)
_CP2 = pltpu.CompilerParams(
    dimension_semantics=("parallel", "arbitrary"),
    vmem_limit_bytes=64 * 1024 * 1024,
)


def _kpool(d_ref, f_ref):
    d = d_ref[0, 0]
    n = jnp.maximum(d, 0.0)
    npad = jnp.pad(n, ((1, 1), (1, 1)))
    v = jnp.maximum(jnp.maximum(npad[0:512], npad[1:513]), npad[2:514])
    v2 = v.reshape(256, 2, 514)[:, 0, :]
    h = jnp.maximum(jnp.maximum(v2[:, 0:512], v2[:, 1:513]), v2[:, 2:514])
    r = jax.lax.broadcasted_iota(jnp.int32, (512, 256), 0)
    c = jax.lax.broadcasted_iota(jnp.int32, (512, 256), 1)
    sel = (r == 2 * c).astype(jnp.float32)
    f_ref[0] = jax.lax.dot(h, sel, preferred_element_type=jnp.float32)


def _k1b(f_ref, w_ref, y_ref, mm_ref, st_ref):
    fv = f_ref[0]
    mask = (fv > 0.0).astype(jnp.float32)

    @pl.when(pl.program_id(1) == 0)
    def _():
        mm_ref[0] = mask

    rolls = [pltpu.roll(fv, (_C0 - (dy * 260 + dx)) % _L, axis=1)
             for dy in range(5) for dx in range(5)]
    X = jnp.concatenate(rolls, axis=0)
    Y = jax.lax.dot(w_ref[0], X, preferred_element_type=jnp.float32)
    Y = Y * mask
    y_ref[0] = Y
    s1 = jnp.sum(Y, axis=1, keepdims=True)
    s2 = jnp.sum(Y * Y, axis=1, keepdims=True)
    cnt = jnp.full((40, 1), jnp.sum(mask), jnp.float32)
    z = jnp.zeros((40, 5), jnp.float32)
    st_ref[0, 0] = jnp.concatenate([s1, s2, cnt, z], axis=1)


def _conv_body(y_ref, mm_ref, ac_ref, w_ref):
    y = y_ref[0]
    mask = mm_ref[0]
    a = ac_ref[:, 0:1]
    c = ac_ref[:, 1:2]
    h = jnp.maximum(a * y + c, 0.0) * mask
    w = w_ref[0]
    cout = w.shape[1]
    acc = jnp.zeros((cout, _L), jnp.float32)
    for t in range(25):
        sh = (_C0 - ((t // 5) * 260 + t % 5)) % _L
        r = pltpu.roll(h, sh, axis=1)
        acc = acc + jax.lax.dot(w[t], r, preferred_element_type=jnp.float32)
    return acc, mask


def _kconv(y_ref, mm_ref, ac_ref, w_ref, out_ref):
    acc, _ = _conv_body(y_ref, mm_ref, ac_ref, w_ref)
    out_ref[0] = acc


def _kconv_st(y_ref, mm_ref, ac_ref, w_ref, out_ref, st_ref):
    acc, mask = _conv_body(y_ref, mm_ref, ac_ref, w_ref)
    out = acc * mask
    out_ref[0] = out
    s1 = jnp.sum(out, axis=1, keepdims=True)
    s2 = jnp.sum(out * out, axis=1, keepdims=True)
    cnt = jnp.full((out.shape[0], 1), jnp.sum(mask), jnp.float32)
    z = jnp.zeros((out.shape[0], 5), jnp.float32)
    st_ref[0, 0] = jnp.concatenate([s1, s2, cnt, z], axis=1)


def _k3b(x1_ref, x2_ref, x3_ref, mm_ref, s_ref, st_ref):
    s = x1_ref[0] + x2_ref[0] + x3_ref[0]
    mask = mm_ref[0]
    s_ref[0] = s
    sm = s * mask
    s1 = jnp.sum(sm, axis=1, keepdims=True)
    s2 = jnp.sum(sm * sm, axis=1, keepdims=True)
    cnt = jnp.full((16, 1), jnp.sum(mask), jnp.float32)
    z = jnp.zeros((16, 5), jnp.float32)
    st_ref[0] = jnp.concatenate([s1, s2, cnt, z], axis=1)


def _k4(s_ref, mm_ref, ac_ref, o_ref):
    s = s_ref[0]
    o_ref[0] = (jnp.maximum(ac_ref[:, 0:1] * s + ac_ref[:, 1:2], 0.0)
                * mm_ref[0])


def _bn_params(s, ss, cnt, g, b, eps=1e-5):
    cnt = jnp.maximum(cnt, 1.0)
    mean = s / cnt
    var = ss / cnt - mean * mean
    a = g * jax.lax.rsqrt(var + eps)
    c = b - mean * a
    return jnp.stack([a, c], axis=1)


def _plane_spec(ch, blk=None, ch_index=None):
    blk = ch if blk is None else blk

    def imap(b, *rest):
        if ch_index is not None:
            return (b, ch_index, 0)
        if rest:
            return (b, rest[0], 0)
        return (b, 0, 0)

    return pl.BlockSpec((1, blk, _L), imap)


def kernel(d, W1, W2a, g2a, b2a, W2b, W3a, g3a, b3a, W3b, g3b, b3b, W3c,
           gf, bf):
    f = pl.pallas_call(
        _kpool,
        grid=(_B,),
        in_specs=[pl.BlockSpec((1, 1, 512, 512), lambda b: (b, 0, 0, 0))],
        out_specs=pl.BlockSpec((1, 256, 256), lambda b: (b, 0, 0)),
        out_shape=jax.ShapeDtypeStruct((_B, 256, 256), jnp.float32),
        compiler_params=_CP,
    )(d)

    fflat = jnp.pad(f, ((0, 0), (2, 2), (2, 2))).reshape(_B, 1, _L)

    w180 = jnp.concatenate([W2a.reshape(25, 32), W3a.reshape(25, 32),
                            W1.reshape(25, 16)], axis=-1)
    w180t = w180.T.reshape(2, 40, 25)

    Y, mm, st1 = pl.pallas_call(
        _k1b,
        grid=(_B, 2),
        in_specs=[
            pl.BlockSpec((1, 1, _L), lambda b, co: (b, 0, 0)),
            pl.BlockSpec((1, 40, 25), lambda b, co: (co, 0, 0)),
        ],
        out_specs=[
            pl.BlockSpec((1, 40, _L), lambda b, co: (b, co, 0)),
            pl.BlockSpec((1, 1, _L), lambda b, co: (b, 0, 0)),
            pl.BlockSpec((1, 1, 40, 8), lambda b, co: (b, co, 0, 0)),
        ],
        out_shape=[
            jax.ShapeDtypeStruct((_B, 80, _L), jnp.float32),
            jax.ShapeDtypeStruct((_B, 1, _L), jnp.float32),
            jax.ShapeDtypeStruct((_B, 2, 40, 8), jnp.float32),
        ],
        compiler_params=_CP2,
    )(fflat, w180t)

    stt = jnp.sum(st1, axis=0)
    cnt = stt[0, 0, 2]
    ac2 = _bn_params(stt[0, 0:32, 0], stt[0, 0:32, 1], cnt, g2a, b2a)
    s_y3 = jnp.concatenate([stt[0, 32:40, 0], stt[1, 0:24, 0]])
    ss_y3 = jnp.concatenate([stt[0, 32:40, 1], stt[1, 0:24, 1]])
    ac3 = _bn_params(s_y3, ss_y3, cnt, g3a, b3a)

    x2 = pl.pallas_call(
        _kconv,
        grid=(_B,),
        in_specs=[
            pl.BlockSpec((1, 32, _L), lambda b: (b, 0, 0)),
            pl.BlockSpec((1, 1, _L), lambda b: (b, 0, 0)),
            pl.BlockSpec((32, 2), lambda b: (0, 0)),
            pl.BlockSpec((1, 25, 16, 32), lambda b: (0, 0, 0, 0)),
        ],
        out_specs=pl.BlockSpec((1, 16, _L), lambda b: (b, 0, 0)),
        out_shape=jax.ShapeDtypeStruct((_B, 16, _L), jnp.float32),
        compiler_params=_CP,
    )(Y, mm, ac2, W2b.reshape(25, 32, 16).transpose(0, 2, 1)[None])

    w3b = W3b.reshape(25, 32, 32).transpose(0, 2, 1)
    w3b = w3b.reshape(25, 2, 16, 32).transpose(1, 0, 2, 3)

    y3b, st3b = pl.pallas_call(
        _kconv_st,
        grid=(_B, 2),
        in_specs=[
            pl.BlockSpec((1, 32, _L), lambda b, co: (b, 1, 0)),
            pl.BlockSpec((1, 1, _L), lambda b, co: (b, 0, 0)),
            pl.BlockSpec((32, 2), lambda b, co: (0, 0)),
            pl.BlockSpec((1, 25, 16, 32), lambda b, co: (co, 0, 0, 0)),
        ],
        out_specs=[
            pl.BlockSpec((1, 16, _L), lambda b, co: (b, co, 0)),
            pl.BlockSpec((1, 1, 16, 8), lambda b, co: (b, co, 0, 0)),
        ],
        out_shape=[
            jax.ShapeDtypeStruct((_B, 32, _L), jnp.float32),
            jax.ShapeDtypeStruct((_B, 2, 16, 8), jnp.float32),
        ],
        compiler_params=_CP2,
    )(Y, mm, ac3, w3b)

    st3 = jnp.sum(st3b, axis=0)
    ac3b = _bn_params(jnp.concatenate([st3[0, :, 0], st3[1, :, 0]]),
                      jnp.concatenate([st3[0, :, 1], st3[1, :, 1]]),
                      cnt, g3b, b3b)

    x3 = pl.pallas_call(
        _kconv,
        grid=(_B,),
        in_specs=[
            pl.BlockSpec((1, 32, _L), lambda b: (b, 0, 0)),
            pl.BlockSpec((1, 1, _L), lambda b: (b, 0, 0)),
            pl.BlockSpec((32, 2), lambda b: (0, 0)),
            pl.BlockSpec((1, 25, 16, 32), lambda b: (0, 0, 0, 0)),
        ],
        out_specs=pl.BlockSpec((1, 16, _L), lambda b: (b, 0, 0)),
        out_shape=jax.ShapeDtypeStruct((_B, 16, _L), jnp.float32),
        compiler_params=_CP,
    )(y3b, mm, ac3b, W3c.reshape(25, 32, 16).transpose(0, 2, 1)[None])

    s, stf = pl.pallas_call(
        _k3b,
        grid=(_B,),
        in_specs=[
            pl.BlockSpec((1, 16, _L), lambda b: (b, 4, 0)),
            pl.BlockSpec((1, 16, _L), lambda b: (b, 0, 0)),
            pl.BlockSpec((1, 16, _L), lambda b: (b, 0, 0)),
            pl.BlockSpec((1, 1, _L), lambda b: (b, 0, 0)),
        ],
        out_specs=[
            pl.BlockSpec((1, 16, _L), lambda b: (b, 0, 0)),
            pl.BlockSpec((1, 16, 8), lambda b: (b, 0, 0)),
        ],
        out_shape=[
            jax.ShapeDtypeStruct((_B, 16, _L), jnp.float32),
            jax.ShapeDtypeStruct((_B, 16, 8), jnp.float32),
        ],
        compiler_params=_CP,
    )(Y, x2, x3, mm)

    stfs = jnp.sum(stf, axis=0)
    acf = _bn_params(stfs[:, 0], stfs[:, 1], cnt, gf, bf)

    o = pl.pallas_call(
        _k4,
        grid=(_B,),
        in_specs=[
            pl.BlockSpec((1, 16, _L), lambda b: (b, 0, 0)),
            pl.BlockSpec((1, 1, _L), lambda b: (b, 0, 0)),
            pl.BlockSpec((16, 2), lambda b: (0, 0)),
        ],
        out_specs=pl.BlockSpec((1, 16, _L), lambda b: (b, 0, 0)),
        out_shape=jax.ShapeDtypeStruct((_B, 16, _L), jnp.float32),
        compiler_params=_CP,
    )(s, mm, acf)

    return o.reshape(_B, 16, 260, 260)[:, :, 2:258, 2:258]


# speedup vs baseline: 3.2746x; 1.0842x over previous
import jax
import jax.numpy as jnp
from jax.experimental import pallas as pl
from jax.experimental.pallas import tpu as pltpu

_B = 4
_L = 68096
_C0 = 522
_CH = 4
_CL = _L // _CH

_CP = pltpu.CompilerParams(
    dimension_semantics=("parallel",),
    vmem_limit_bytes=64 * 1024 * 1024,
)
_CP2 = pltpu.CompilerParams(
    dimension_semantics=("parallel", "arbitrary"),
    vmem_limit_bytes=64 * 1024 * 1024,
)


def _kpool(d_ref, f_ref):
    d = d_ref[0, 0]
    n = jnp.maximum(d, 0.0)
    npad = jnp.pad(n, ((1, 1), (1, 1)))
    v = jnp.maximum(jnp.maximum(npad[0:512], npad[1:513]), npad[2:514])
    v2 = v.reshape(256, 2, 514)[:, 0, :]
    h = jnp.maximum(jnp.maximum(v2[:, 0:512], v2[:, 1:513]), v2[:, 2:514])
    r = jax.lax.broadcasted_iota(jnp.int32, (512, 256), 0)
    c = jax.lax.broadcasted_iota(jnp.int32, (512, 256), 1)
    sel = (r == 2 * c).astype(jnp.float32)
    f_ref[0] = jax.lax.dot(h, sel, preferred_element_type=jnp.float32)


def _k1b(f_ref, w_ref, y_ref, mm_ref, st_ref):
    fv = f_ref[0]
    mask = (fv > 0.0).astype(jnp.float32)

    @pl.when(pl.program_id(1) == 0)
    def _():
        mm_ref[0] = mask

    rolls = [pltpu.roll(fv, (_C0 - (dy * 260 + dx)) % _L, axis=1)
             for dy in range(5) for dx in range(5)]
    X = jnp.concatenate(rolls, axis=0)
    Y = jax.lax.dot(w_ref[0], X, preferred_element_type=jnp.float32)
    Y = Y * mask
    y_ref[0] = Y
    s1 = jnp.sum(Y, axis=1, keepdims=True)
    s2 = jnp.sum(Y * Y, axis=1, keepdims=True)
    cnt = jnp.full((40, 1), jnp.sum(mask), jnp.float32)
    z = jnp.zeros((40, 5), jnp.float32)
    st_ref[0, 0] = jnp.concatenate([s1, s2, cnt, z], axis=1)


def _conv_chunk(hp, w, n):
    cout = w.shape[1]
    sub = jnp.zeros((cout, _CL), jnp.float32)
    for dy in range(5):
        base = dy * 260 + n * _CL
        Xc = jnp.concatenate([hp[:, base + dx:base + dx + _CL]
                              for dx in range(5)], axis=0)
        sub = sub + jax.lax.dot(w[dy], Xc,
                                preferred_element_type=jnp.float32)
    return sub


def _prep_h(y_ref, mm_ref, ac_ref):
    y = y_ref[0]
    mask = mm_ref[0]
    a = ac_ref[:, 0:1]
    c = ac_ref[:, 1:2]
    h = jnp.maximum(a * y + c, 0.0) * mask
    return jnp.pad(h, ((0, 0), (_C0, _C0))), mask


def _kconv(y_ref, mm_ref, ac_ref, w_ref, out_ref):
    hp, _ = _prep_h(y_ref, mm_ref, ac_ref)
    w = w_ref[0]
    for n in range(_CH):
        out_ref[0, :, n * _CL:(n + 1) * _CL] = _conv_chunk(hp, w, n)


def _kconv_st(y_ref, mm_ref, ac_ref, w_ref, out_ref, st_ref):
    hp, mask = _prep_h(y_ref, mm_ref, ac_ref)
    w = w_ref[0]
    cout = w.shape[1]
    s1 = jnp.zeros((cout, 1), jnp.float32)
    s2 = jnp.zeros((cout, 1), jnp.float32)
    for n in range(_CH):
        outc = _conv_chunk(hp, w, n) * mask[:, n * _CL:(n + 1) * _CL]
        out_ref[0, :, n * _CL:(n + 1) * _CL] = outc
        s1 = s1 + jnp.sum(outc, axis=1, keepdims=True)
        s2 = s2 + jnp.sum(outc * outc, axis=1, keepdims=True)
    cnt = jnp.full((cout, 1), jnp.sum(mask), jnp.float32)
    z = jnp.zeros((cout, 5), jnp.float32)
    st_ref[0, 0] = jnp.concatenate([s1, s2, cnt, z], axis=1)


def _k3b(x1_ref, x2_ref, x3_ref, mm_ref, s_ref, st_ref):
    s = x1_ref[0] + x2_ref[0] + x3_ref[0]
    mask = mm_ref[0]
    s_ref[0] = s
    sm = s * mask
    s1 = jnp.sum(sm, axis=1, keepdims=True)
    s2 = jnp.sum(sm * sm, axis=1, keepdims=True)
    cnt = jnp.full((16, 1), jnp.sum(mask), jnp.float32)
    z = jnp.zeros((16, 5), jnp.float32)
    st_ref[0] = jnp.concatenate([s1, s2, cnt, z], axis=1)


def _k4(s_ref, mm_ref, ac_ref, o_ref):
    s = s_ref[0]
    o_ref[0] = (jnp.maximum(ac_ref[:, 0:1] * s + ac_ref[:, 1:2], 0.0)
                * mm_ref[0])


def _bn_params(s, ss, cnt, g, b, eps=1e-5):
    cnt = jnp.maximum(cnt, 1.0)
    mean = s / cnt
    var = ss / cnt - mean * mean
    a = g * jax.lax.rsqrt(var + eps)
    c = b - mean * a
    return jnp.stack([a, c], axis=1)


def _plane_spec(ch, blk=None, ch_index=None):
    blk = ch if blk is None else blk

    def imap(b, *rest):
        if ch_index is not None:
            return (b, ch_index, 0)
        if rest:
            return (b, rest[0], 0)
        return (b, 0, 0)

    return pl.BlockSpec((1, blk, _L), imap)


def kernel(d, W1, W2a, g2a, b2a, W2b, W3a, g3a, b3a, W3b, g3b, b3b, W3c,
           gf, bf):
    f = pl.pallas_call(
        _kpool,
        grid=(_B,),
        in_specs=[pl.BlockSpec((1, 1, 512, 512), lambda b: (b, 0, 0, 0))],
        out_specs=pl.BlockSpec((1, 256, 256), lambda b: (b, 0, 0)),
        out_shape=jax.ShapeDtypeStruct((_B, 256, 256), jnp.float32),
        compiler_params=_CP,
    )(d)

    fflat = jnp.pad(f, ((0, 0), (2, 2), (2, 2))).reshape(_B, 1, 67600)
    fflat = jnp.pad(fflat, ((0, 0), (0, 0), (0, _L - 67600)))

    w180 = jnp.concatenate([W2a.reshape(25, 32), W3a.reshape(25, 32),
                            W1.reshape(25, 16)], axis=-1)
    w180t = w180.T.reshape(2, 40, 25)

    Y, mm, st1 = pl.pallas_call(
        _k1b,
        grid=(_B, 2),
        in_specs=[
            pl.BlockSpec((1, 1, _L), lambda b, co: (b, 0, 0)),
            pl.BlockSpec((1, 40, 25), lambda b, co: (co, 0, 0)),
        ],
        out_specs=[
            pl.BlockSpec((1, 40, _L), lambda b, co: (b, co, 0)),
            pl.BlockSpec((1, 1, _L), lambda b, co: (b, 0, 0)),
            pl.BlockSpec((1, 1, 40, 8), lambda b, co: (b, co, 0, 0)),
        ],
        out_shape=[
            jax.ShapeDtypeStruct((_B, 80, _L), jnp.float32),
            jax.ShapeDtypeStruct((_B, 1, _L), jnp.float32),
            jax.ShapeDtypeStruct((_B, 2, 40, 8), jnp.float32),
        ],
        compiler_params=_CP2,
    )(fflat, w180t)

    stt = jnp.sum(st1, axis=0)
    cnt = stt[0, 0, 2]
    ac2 = _bn_params(stt[0, 0:32, 0], stt[0, 0:32, 1], cnt, g2a, b2a)
    s_y3 = jnp.concatenate([stt[0, 32:40, 0], stt[1, 0:24, 0]])
    ss_y3 = jnp.concatenate([stt[0, 32:40, 1], stt[1, 0:24, 1]])
    ac3 = _bn_params(s_y3, ss_y3, cnt, g3a, b3a)

    x2 = pl.pallas_call(
        _kconv,
        grid=(_B,),
        in_specs=[
            pl.BlockSpec((1, 32, _L), lambda b: (b, 0, 0)),
            pl.BlockSpec((1, 1, _L), lambda b: (b, 0, 0)),
            pl.BlockSpec((32, 2), lambda b: (0, 0)),
            pl.BlockSpec((1, 5, 16, 160), lambda b: (0, 0, 0, 0)),
        ],
        out_specs=pl.BlockSpec((1, 16, _L), lambda b: (b, 0, 0)),
        out_shape=jax.ShapeDtypeStruct((_B, 16, _L), jnp.float32),
        compiler_params=_CP,
    )(Y, mm, ac2, W2b.transpose(0, 3, 1, 2).reshape(5, 16, 160)[None])

    w3b = W3b.transpose(0, 3, 1, 2).reshape(5, 2, 16, 160).transpose(1, 0, 2, 3)

    y3b, st3b = pl.pallas_call(
        _kconv_st,
        grid=(_B, 2),
        in_specs=[
            pl.BlockSpec((1, 32, _L), lambda b, co: (b, 1, 0)),
            pl.BlockSpec((1, 1, _L), lambda b, co: (b, 0, 0)),
            pl.BlockSpec((32, 2), lambda b, co: (0, 0)),
            pl.BlockSpec((1, 5, 16, 160), lambda b, co: (co, 0, 0, 0)),
        ],
        out_specs=[
            pl.BlockSpec((1, 16, _L), lambda b, co: (b, co, 0)),
            pl.BlockSpec((1, 1, 16, 8), lambda b, co: (b, co, 0, 0)),
        ],
        out_shape=[
            jax.ShapeDtypeStruct((_B, 32, _L), jnp.float32),
            jax.ShapeDtypeStruct((_B, 2, 16, 8), jnp.float32),
        ],
        compiler_params=_CP2,
    )(Y, mm, ac3, w3b)

    st3 = jnp.sum(st3b, axis=0)
    ac3b = _bn_params(jnp.concatenate([st3[0, :, 0], st3[1, :, 0]]),
                      jnp.concatenate([st3[0, :, 1], st3[1, :, 1]]),
                      cnt, g3b, b3b)

    x3 = pl.pallas_call(
        _kconv,
        grid=(_B,),
        in_specs=[
            pl.BlockSpec((1, 32, _L), lambda b: (b, 0, 0)),
            pl.BlockSpec((1, 1, _L), lambda b: (b, 0, 0)),
            pl.BlockSpec((32, 2), lambda b: (0, 0)),
            pl.BlockSpec((1, 5, 16, 160), lambda b: (0, 0, 0, 0)),
        ],
        out_specs=pl.BlockSpec((1, 16, _L), lambda b: (b, 0, 0)),
        out_shape=jax.ShapeDtypeStruct((_B, 16, _L), jnp.float32),
        compiler_params=_CP,
    )(y3b, mm, ac3b, W3c.transpose(0, 3, 1, 2).reshape(5, 16, 160)[None])

    s, stf = pl.pallas_call(
        _k3b,
        grid=(_B,),
        in_specs=[
            pl.BlockSpec((1, 16, _L), lambda b: (b, 4, 0)),
            pl.BlockSpec((1, 16, _L), lambda b: (b, 0, 0)),
            pl.BlockSpec((1, 16, _L), lambda b: (b, 0, 0)),
            pl.BlockSpec((1, 1, _L), lambda b: (b, 0, 0)),
        ],
        out_specs=[
            pl.BlockSpec((1, 16, _L), lambda b: (b, 0, 0)),
            pl.BlockSpec((1, 16, 8), lambda b: (b, 0, 0)),
        ],
        out_shape=[
            jax.ShapeDtypeStruct((_B, 16, _L), jnp.float32),
            jax.ShapeDtypeStruct((_B, 16, 8), jnp.float32),
        ],
        compiler_params=_CP,
    )(Y, x2, x3, mm)

    stfs = jnp.sum(stf, axis=0)
    acf = _bn_params(stfs[:, 0], stfs[:, 1], cnt, gf, bf)

    o = pl.pallas_call(
        _k4,
        grid=(_B,),
        in_specs=[
            pl.BlockSpec((1, 16, _L), lambda b: (b, 0, 0)),
            pl.BlockSpec((1, 1, _L), lambda b: (b, 0, 0)),
            pl.BlockSpec((16, 2), lambda b: (0, 0)),
        ],
        out_specs=pl.BlockSpec((1, 16, _L), lambda b: (b, 0, 0)),
        out_shape=jax.ShapeDtypeStruct((_B, 16, _L), jnp.float32),
        compiler_params=_CP,
    )(s, mm, acf)

    return o[:, :, :67600].reshape(_B, 16, 260, 260)[:, :, 2:258, 2:258]
